# baseline (speedup 1.0000x reference)
import functools

import jax
import jax.numpy as jnp
from jax import lax
from jax.experimental import pallas as pl
from jax.experimental.pallas import tpu as pltpu
from jax.experimental.pallas import tpu_sc as plsc

M = 1024
N = 100000
D = 32
B = 16384

_NC = 2
_NS = 16
_NW = _NC * _NS
_BPW = B // _NW
_CH = 128
_NCH = _BPW // _CH
_ROWS_PER_W = B // _CH // _NW

_KB = 4096
_NKB = -(-N // _KB)


def _sc_body(i_hbm, j_hbm, w_hbm, b_hbm, zeros_hbm,
             a2_out, bj_out,
             iidx, jidx, rows, bjv, bounce, shared_a):
    c = lax.axis_index("c")
    s = lax.axis_index("s")
    wid = s * _NC + c
    base = wid * _ROWS_PER_W

    pltpu.sync_copy(i_hbm.at[pl.ds(base, _ROWS_PER_W)], iidx)
    pltpu.sync_copy(j_hbm.at[pl.ds(base, _ROWS_PER_W)], jidx)

    @pl.when(s == 0)
    def _zero():
        pltpu.sync_copy(zeros_hbm, shared_a)

    plsc.subcore_barrier()

    for k in range(_NCH):
        pltpu.sync_copy(w_hbm.at[jidx.at[k]], rows.at[pl.ds(k * _CH, _CH)])
        pltpu.sync_copy(b_hbm.at[jidx.at[k]], bjv.at[k])

    for k in range(_NCH):
        pltpu.sync_copy(rows.at[pl.ds(k * _CH, _CH)],
                        shared_a.at[iidx.at[k]], add=True)

    pltpu.sync_copy(bjv, bj_out.at[pl.ds(base, _ROWS_PER_W)])

    plsc.subcore_barrier()

    @pl.when(s == 0)
    def _publish():
        pltpu.sync_copy(shared_a, bounce)
        pltpu.sync_copy(bounce, a2_out.at[c])


@functools.cache
def _sc_call():
    return pl.kernel(
        _sc_body,
        out_type=[
            jax.ShapeDtypeStruct((_NC, M, D), jnp.float32),
            jax.ShapeDtypeStruct((B // _CH, _CH), jnp.float32),
        ],
        mesh=plsc.VectorSubcoreMesh(
            core_axis_name="c", subcore_axis_name="s", num_cores=_NC),
        scratch_types=[
            pltpu.VMEM((_ROWS_PER_W, _CH), jnp.int32),
            pltpu.VMEM((_ROWS_PER_W, _CH), jnp.int32),
            pltpu.VMEM((_BPW, D), jnp.float32),
            pltpu.VMEM((_ROWS_PER_W, _CH), jnp.float32),
            pltpu.VMEM((M, D), jnp.float32),
            pltpu.VMEM_SHARED((M, D), jnp.float32),
        ],
        compiler_params=pltpu.CompilerParams(use_tc_tiling_on_sc=False),
    )


def _tc_body(r_ref, v_ref, mu_ref, a2_ref, bj_ref, out_ref, acc_ref):
    k = pl.program_id(0)

    @pl.when(k == 0)
    def _init():
        acc_ref[...] = jnp.zeros_like(acc_ref)

    base = k * _KB
    lane = jax.lax.broadcasted_iota(jnp.int32, (M, _KB), 1) + base
    r_blk = jnp.where(lane < N, r_ref[...], 0.0)
    sub = jax.lax.broadcasted_iota(jnp.int32, (_KB, D), 0) + base
    v_blk = jnp.where(sub < N, v_ref[...], 0.0)
    acc_ref[...] += jnp.dot(r_blk, v_blk, preferred_element_type=jnp.float32)

    @pl.when(k == pl.num_programs(0) - 1)
    def _epilogue():
        h = jax.nn.sigmoid(acc_ref[...] + mu_ref[...])
        a = a2_ref[0] + a2_ref[1]
        s = jnp.sum(h * a)
        out_ref[...] = s + bj_ref[...]


def _tc_call(r, v, mu, a2, bj):
    return pl.pallas_call(
        _tc_body,
        grid=(_NKB,),
        in_specs=[
            pl.BlockSpec((M, _KB), lambda k: (0, k)),
            pl.BlockSpec((_KB, D), lambda k: (k, 0)),
            pl.BlockSpec((1, D), lambda k: (0, 0)),
            pl.BlockSpec((_NC, M, D), lambda k: (0, 0, 0)),
            pl.BlockSpec((B // _CH, _CH), lambda k: (0, 0)),
        ],
        out_specs=pl.BlockSpec((B // _CH, _CH), lambda k: (0, 0)),
        out_shape=jax.ShapeDtypeStruct((B // _CH, _CH), jnp.float32),
        scratch_shapes=[pltpu.VMEM((M, D), jnp.float32)],
        compiler_params=pltpu.CompilerParams(
            dimension_semantics=("arbitrary",),
        ),
    )(r, v, mu, a2, bj)


def kernel(r, i, j, v, mu, w, b):
    i2 = i.astype(jnp.int32).reshape(B // _CH, _CH)
    j2 = j.astype(jnp.int32).reshape(B // _CH, _CH)
    zeros = jnp.zeros((M, D), jnp.float32)
    a2, bj = _sc_call()(i2, j2, w, b, zeros)
    out2 = _tc_call(r, v, mu, a2, bj)
    return out2.reshape(B)


# speedup vs baseline: 3.0073x; 3.0073x over previous
import functools

import jax
import jax.numpy as jnp
from jax import lax
from jax.experimental import pallas as pl
from jax.experimental.pallas import tpu as pltpu
from jax.experimental.pallas import tpu_sc as plsc

M = 1024
N = 100000
D = 32
B = 16384

_NC = 2
_NS = 16
_NW = _NC * _NS
_BPW = B // _NW
_CH = 128
_NCH = _BPW // _CH
_ROWS_PER_W = B // _CH // _NW

_KB = 4096
_NKB = -(-N // _KB)


def _sc_body(i_hbm, j_hbm, w_hbm, b_hbm, zeros_hbm,
             a2_out, bj_out,
             iidx, jidx, rows, bjv, bounce, shared_a):
    c = lax.axis_index("c")
    s = lax.axis_index("s")
    wid = s * _NC + c
    base = wid * _ROWS_PER_W

    pltpu.sync_copy(i_hbm.at[pl.ds(base, _ROWS_PER_W)], iidx)
    pltpu.sync_copy(j_hbm.at[pl.ds(base, _ROWS_PER_W)], jidx)

    @pl.when(s == 0)
    def _zero():
        pltpu.sync_copy(zeros_hbm, shared_a)

    plsc.subcore_barrier()

    for k in range(_NCH):
        pltpu.sync_copy(w_hbm.at[jidx.at[k]], rows.at[pl.ds(k * _CH, _CH)])
        pltpu.sync_copy(b_hbm.at[jidx.at[k]], bjv.at[k])

    for k in range(_NCH):
        pltpu.sync_copy(rows.at[pl.ds(k * _CH, _CH)],
                        shared_a.at[iidx.at[k]], add=True)

    pltpu.sync_copy(bjv, bj_out.at[pl.ds(base, _ROWS_PER_W)])

    plsc.subcore_barrier()

    @pl.when(s == 0)
    def _publish():
        pltpu.sync_copy(shared_a, bounce)
        pltpu.sync_copy(bounce, a2_out.at[c])


@functools.cache
def _sc_call():
    return pl.kernel(
        _sc_body,
        out_type=[
            jax.ShapeDtypeStruct((_NC, M, D), jnp.float32),
            jax.ShapeDtypeStruct((B // _CH, _CH), jnp.float32),
        ],
        mesh=plsc.VectorSubcoreMesh(
            core_axis_name="c", subcore_axis_name="s", num_cores=_NC),
        scratch_types=[
            pltpu.VMEM((_ROWS_PER_W, _CH), jnp.int32),
            pltpu.VMEM((_ROWS_PER_W, _CH), jnp.int32),
            pltpu.VMEM((_BPW, D), jnp.float32),
            pltpu.VMEM((_ROWS_PER_W, _CH), jnp.float32),
            pltpu.VMEM((M, D), jnp.float32),
            pltpu.VMEM_SHARED((M, D), jnp.float32),
        ],
        compiler_params=pltpu.CompilerParams(use_tc_tiling_on_sc=False),
    )


def _mm_body(vt_ref, rt_ref, mut_ref, ht_ref, acc_ref):
    k = pl.program_id(0)

    @pl.when(k == 0)
    def _init():
        acc_ref[...] = jnp.zeros_like(acc_ref)

    base = k * _KB
    rows = lax.broadcasted_iota(jnp.int32, (_KB, M), 0) + base
    rt = jnp.where(rows < N, rt_ref[...], 0.0)
    cols = lax.broadcasted_iota(jnp.int32, (D, _KB), 1) + base
    vt = jnp.where(cols < N, vt_ref[...], 0.0)
    acc_ref[...] += jnp.dot(vt, rt, preferred_element_type=jnp.float32)

    @pl.when(k == pl.num_programs(0) - 1)
    def _epilogue():
        ht_ref[...] = jax.nn.sigmoid(acc_ref[...] + mut_ref[...])


def _mm_call(vt, rt, mut):
    return pl.pallas_call(
        _mm_body,
        grid=(_NKB,),
        in_specs=[
            pl.BlockSpec((D, _KB), lambda k: (0, k)),
            pl.BlockSpec((_KB, M), lambda k: (k, 0)),
            pl.BlockSpec((D, 1), lambda k: (0, 0)),
        ],
        out_specs=pl.BlockSpec((D, M), lambda k: (0, 0)),
        out_shape=jax.ShapeDtypeStruct((D, M), jnp.float32),
        scratch_shapes=[pltpu.VMEM((D, M), jnp.float32)],
        compiler_params=pltpu.CompilerParams(
            dimension_semantics=("arbitrary",),
        ),
    )(vt, rt, mut)


def _combine_body(ht_ref, a2_ref, bj_ref, out_ref):
    a = a2_ref[0] + a2_ref[1]
    mm = jnp.dot(ht_ref[...], a, preferred_element_type=jnp.float32,
                 precision=lax.Precision.HIGHEST)
    row = lax.broadcasted_iota(jnp.int32, (D, D), 0)
    col = lax.broadcasted_iota(jnp.int32, (D, D), 1)
    s = jnp.sum(jnp.where(row == col, mm, 0.0))
    out_ref[...] = s + bj_ref[...]


def _combine_call(ht, a2, bj):
    return pl.pallas_call(
        _combine_body,
        in_specs=[
            pl.BlockSpec((D, M), lambda: (0, 0)),
            pl.BlockSpec((_NC, M, D), lambda: (0, 0, 0)),
            pl.BlockSpec((B // _CH, _CH), lambda: (0, 0)),
        ],
        out_specs=pl.BlockSpec((B // _CH, _CH), lambda: (0, 0)),
        out_shape=jax.ShapeDtypeStruct((B // _CH, _CH), jnp.float32),
    )(ht, a2, bj)


def kernel(r, i, j, v, mu, w, b):
    i2 = i.astype(jnp.int32).reshape(B // _CH, _CH)
    j2 = j.astype(jnp.int32).reshape(B // _CH, _CH)
    zeros = jnp.zeros((M, D), jnp.float32)
    a2, bj = _sc_call()(i2, j2, w, b, zeros)
    ht = _mm_call(v.T, r.T, mu.T)
    out2 = _combine_call(ht, a2, bj)
    return out2.reshape(B)
